# Initial kernel scaffold; baseline (speedup 1.0000x reference)
import jax
import jax.numpy as jnp
from jax.experimental import pallas as pl


def kernel(x, edge_index, W1_self, W1_neigh, b1, W2_self, W2_neigh, b2):
    raise NotImplementedError("write your pallas kernel here")


# speedup vs baseline: 7.6832x; 7.6832x over previous
import functools

import jax
import jax.numpy as jnp
from jax import lax
from jax.experimental import pallas as pl
from jax.experimental.pallas import tpu as pltpu
from jax.experimental.pallas import tpu_sc as plsc

NC = 2
NS = 16
NW = NC * NS

_CHUNK = 80


def _sc_agg(table, src3, dst3, z_feat, z_deg, *, with_deg):
    n, d = table.shape
    nchunk = src3.shape[1]
    rows_a = ((n + NS - 1) // NS + 7) // 8 * 8
    rows_last = n - rows_a * (NS - 1)
    mesh = plsc.VectorSubcoreMesh(
        core_axis_name="c", subcore_axis_name="s", num_cores=NC,
        num_subcores=NS)

    out_type = [jax.ShapeDtypeStruct((NC, n, d), jnp.float32)]
    scratch = [
        pltpu.VMEM((nchunk, _CHUNK), jnp.int32),
        pltpu.VMEM((nchunk, _CHUNK), jnp.int32),
        pltpu.VMEM((_CHUNK, d), jnp.float32),
        pltpu.VMEM_SHARED((n, d), jnp.float32),
    ]
    if with_deg:
        out_type.append(jax.ShapeDtypeStruct((NC, n, 16), jnp.float32))
        scratch += [
            pltpu.VMEM((_CHUNK, 16), jnp.float32),
            pltpu.VMEM_SHARED((n, 16), jnp.float32),
        ]

    @functools.partial(pl.kernel, out_type=out_type, mesh=mesh,
                       scratch_types=scratch,
                       compiler_params=pltpu.CompilerParams(
                           use_tc_tiling_on_sc=False))
    def k(table_hbm, src_hbm, dst_hbm, zf_hbm, zd_hbm, *refs):
        if with_deg:
            (agg_out, deg_out, src_v, dst_v, rows_v, agg_sh, ones_v,
             deg_sh) = refs
        else:
            agg_out, src_v, dst_v, rows_v, agg_sh = refs
        cid = lax.axis_index("c")
        sid = lax.axis_index("s")
        wid = cid * NS + sid

        pltpu.sync_copy(src_hbm.at[wid], src_v)
        pltpu.sync_copy(dst_hbm.at[wid], dst_v)
        row0 = sid * rows_a

        def zero_slice(nrows):
            sl = pl.ds(row0, nrows)
            pltpu.sync_copy(zf_hbm.at[sl], agg_sh.at[sl])
            if with_deg:
                pltpu.sync_copy(zd_hbm.at[sl], deg_sh.at[sl])

        pl.when(sid < NS - 1)(lambda: zero_slice(rows_a))
        pl.when(sid == NS - 1)(lambda: zero_slice(rows_last))
        if with_deg:

            @pl.loop(0, _CHUNK)
            def _(r):
                ones_v[r, :] = jnp.ones((16,), jnp.float32)

        plsc.subcore_barrier()

        @pl.loop(0, nchunk)
        def _(j):
            pltpu.sync_copy(table_hbm.at[src_v.at[j]], rows_v)
            pltpu.sync_copy(rows_v, agg_sh.at[dst_v.at[j]], add=True)
            if with_deg:
                pltpu.sync_copy(ones_v, deg_sh.at[dst_v.at[j]], add=True)

        plsc.subcore_barrier()

        def copy_out(nrows):
            sl = pl.ds(row0, nrows)
            pltpu.sync_copy(agg_sh.at[sl], agg_out.at[cid, sl])
            if with_deg:
                pltpu.sync_copy(deg_sh.at[sl], deg_out.at[cid, sl])

        pl.when(sid < NS - 1)(lambda: copy_out(rows_a))
        pl.when(sid == NS - 1)(lambda: copy_out(rows_last))

    return k(table, src3, dst3, z_feat, z_deg)


def _tc_dense(x, agg, deg, w_self, w_neigh, b, *, relu):
    n, d = x.shape
    r = 1000

    def body(x_ref, agg_ref, deg_ref, ws_ref, wn_ref, b_ref, o_ref):
        degs = deg_ref[0] + deg_ref[1]
        inv = 1.0 / jnp.maximum(degs[:, :1], 1.0)
        mean = (agg_ref[0] + agg_ref[1]) * inv
        out = (jnp.dot(x_ref[...], ws_ref[...],
                       preferred_element_type=jnp.float32)
               + jnp.dot(mean, wn_ref[...],
                         preferred_element_type=jnp.float32)
               + b_ref[...])
        o_ref[...] = jnp.maximum(out, 0.0) if relu else out

    return pl.pallas_call(
        body,
        grid=(n // r,),
        in_specs=[
            pl.BlockSpec((r, d), lambda i: (i, 0)),
            pl.BlockSpec((NC, r, d), lambda i: (0, i, 0)),
            pl.BlockSpec((NC, r, 16), lambda i: (0, i, 0)),
            pl.BlockSpec((d, d), lambda i: (0, 0)),
            pl.BlockSpec((d, d), lambda i: (0, 0)),
            pl.BlockSpec((1, d), lambda i: (0, 0)),
        ],
        out_specs=pl.BlockSpec((r, d), lambda i: (i, 0)),
        out_shape=jax.ShapeDtypeStruct((n, d), jnp.float32),
    )(x, agg, deg, w_self, w_neigh, b.reshape(1, d))


def kernel(x, edge_index, W1_self, W1_neigh, b1, W2_self, W2_neigh, b2):
    n, d = x.shape
    e = edge_index.shape[1]
    e_per_w = e // NW
    nchunk = e_per_w // _CHUNK
    src3 = edge_index[0].reshape(NW, nchunk, _CHUNK)
    dst3 = edge_index[1].reshape(NW, nchunk, _CHUNK)
    z_feat = jnp.zeros((n, d), jnp.float32)
    z_deg = jnp.zeros((n, 16), jnp.float32)

    agg1, deg = _sc_agg(x, src3, dst3, z_feat, z_deg, with_deg=True)
    h = _tc_dense(x, agg1, deg, W1_self, W1_neigh, b1, relu=True)
    (agg2,) = _sc_agg(h, src3, dst3, z_feat, z_deg, with_deg=False)
    out = _tc_dense(h, agg2, deg, W2_self, W2_neigh, b2, relu=False)
    return out


# speedup vs baseline: 12.5251x; 1.6302x over previous
import functools

import jax
import jax.numpy as jnp
from jax import lax
from jax.experimental import pallas as pl
from jax.experimental.pallas import tpu as pltpu
from jax.experimental.pallas import tpu_sc as plsc

NC = 2
NS = 16
NW = NC * NS

_CHUNK = 100


def _sc_agg(table, src4, dst3, z_feat, z_deg, *, with_deg):
    n, d = table.shape
    ngroup = src4.shape[1]
    nchunk = dst3.shape[1]
    rows_a = ((n + NS - 1) // NS + 7) // 8 * 8
    rows_last = n - rows_a * (NS - 1)
    mesh = plsc.VectorSubcoreMesh(
        core_axis_name="c", subcore_axis_name="s", num_cores=NC,
        num_subcores=NS)

    out_type = [jax.ShapeDtypeStruct((NC, n, d), jnp.float32)]
    scratch = [
        pltpu.VMEM((2, 2, _CHUNK), jnp.int32),
        pltpu.VMEM((nchunk, _CHUNK), jnp.int32),
        pltpu.VMEM((2, _CHUNK, d), jnp.float32),
        [pltpu.SemaphoreType.DMA] * 2,
        [pltpu.SemaphoreType.DMA] * 2,
        [pltpu.SemaphoreType.DMA] * 2,
        pltpu.VMEM_SHARED((n, d), jnp.float32),
    ]
    if with_deg:
        out_type.append(jax.ShapeDtypeStruct((NC, n, 16), jnp.float32))
        scratch += [
            pltpu.VMEM((_CHUNK, 16), jnp.float32),
            pltpu.VMEM_SHARED((n, 16), jnp.float32),
        ]

    @functools.partial(pl.kernel, out_type=out_type, mesh=mesh,
                       scratch_types=scratch,
                       compiler_params=pltpu.CompilerParams(
                           use_tc_tiling_on_sc=False))
    def k(table_hbm, src_hbm, dst_hbm, zf_hbm, zd_hbm, *refs):
        if with_deg:
            (agg_out, deg_out, sidx, dst_v, rows_v, gsem, ssem, isem,
             agg_sh, ones_v, deg_sh) = refs
        else:
            agg_out, sidx, dst_v, rows_v, gsem, ssem, isem, agg_sh = refs
        cid = lax.axis_index("c")
        sid = lax.axis_index("s")
        wid = cid * NS + sid

        pltpu.sync_copy(dst_hbm.at[wid], dst_v)
        pltpu.sync_copy(src_hbm.at[wid, 0], sidx.at[0])
        pltpu.async_copy(src_hbm.at[wid, 1], sidx.at[1], isem[1])
        row0 = sid * rows_a

        def zero_slice(nrows):
            sl = pl.ds(row0, nrows)
            pltpu.sync_copy(zf_hbm.at[sl], agg_sh.at[sl])
            if with_deg:
                pltpu.sync_copy(zd_hbm.at[sl], deg_sh.at[sl])

        pl.when(sid < NS - 1)(lambda: zero_slice(rows_a))
        pl.when(sid == NS - 1)(lambda: zero_slice(rows_last))
        if with_deg:

            @pl.loop(0, _CHUNK)
            def _(r):
                ones_v[r, :] = jnp.ones((16,), jnp.float32)

        for b in range(2):
            pltpu.async_copy(table_hbm.at[sidx.at[0, b]], rows_v.at[b],
                             gsem[b])

        plsc.subcore_barrier()

        def step(g, G, H, b):
            j = g * 2 + b
            pltpu.make_async_copy(table_hbm.at[sidx.at[0, b]],
                                  rows_v.at[b], gsem[b]).wait()
            if b == 0:
                @pl.when(g < ngroup - 1)
                def _():
                    pltpu.make_async_copy(src_hbm.at[wid, 0], sidx.at[H],
                                          isem[H]).wait()
            pltpu.async_copy(rows_v.at[b], agg_sh.at[dst_v.at[j]],
                             ssem[b], add=True)
            if with_deg:
                pltpu.sync_copy(ones_v, deg_sh.at[dst_v.at[j]], add=True)
            if b == 1:
                @pl.when(g < ngroup - 2)
                def _():
                    pltpu.async_copy(src_hbm.at[wid, g + 2], sidx.at[G],
                                     isem[G])

            def drain_scatter():
                pltpu.make_async_copy(rows_v.at[b],
                                      agg_sh.at[dst_v.at[j]],
                                      ssem[b]).wait()

            @pl.when(g < ngroup - 1)
            def _():
                drain_scatter()
                pltpu.async_copy(table_hbm.at[sidx.at[H, b]],
                                 rows_v.at[b], gsem[b])

            pl.when(g == ngroup - 1)(drain_scatter)

        @pl.loop(0, ngroup // 2)
        def _(t):
            for b in range(2):
                step(2 * t, 0, 1, b)
            for b in range(2):
                step(2 * t + 1, 1, 0, b)

        plsc.subcore_barrier()

        def copy_out(nrows):
            sl = pl.ds(row0, nrows)
            pltpu.sync_copy(agg_sh.at[sl], agg_out.at[cid, sl])
            if with_deg:
                pltpu.sync_copy(deg_sh.at[sl], deg_out.at[cid, sl])

        pl.when(sid < NS - 1)(lambda: copy_out(rows_a))
        pl.when(sid == NS - 1)(lambda: copy_out(rows_last))

    return k(table, src4, dst3, z_feat, z_deg)


def _tc_dense(x, agg, deg, w_self, w_neigh, b, *, relu):
    n, d = x.shape
    r = 1000

    def body(x_ref, agg_ref, deg_ref, ws_ref, wn_ref, b_ref, o_ref):
        degs = deg_ref[0] + deg_ref[1]
        inv = 1.0 / jnp.maximum(degs[:, :1], 1.0)
        mean = (agg_ref[0] + agg_ref[1]) * inv
        out = (jnp.dot(x_ref[...], ws_ref[...],
                       preferred_element_type=jnp.float32)
               + jnp.dot(mean, wn_ref[...],
                         preferred_element_type=jnp.float32)
               + b_ref[...])
        o_ref[...] = jnp.maximum(out, 0.0) if relu else out

    return pl.pallas_call(
        body,
        grid=(n // r,),
        in_specs=[
            pl.BlockSpec((r, d), lambda i: (i, 0)),
            pl.BlockSpec((NC, r, d), lambda i: (0, i, 0)),
            pl.BlockSpec((NC, r, 16), lambda i: (0, i, 0)),
            pl.BlockSpec((d, d), lambda i: (0, 0)),
            pl.BlockSpec((d, d), lambda i: (0, 0)),
            pl.BlockSpec((1, d), lambda i: (0, 0)),
        ],
        out_specs=pl.BlockSpec((r, d), lambda i: (i, 0)),
        out_shape=jax.ShapeDtypeStruct((n, d), jnp.float32),
    )(x, agg, deg, w_self, w_neigh, b.reshape(1, d))


def kernel(x, edge_index, W1_self, W1_neigh, b1, W2_self, W2_neigh, b2):
    n, d = x.shape
    e = edge_index.shape[1]
    e_per_w = e // NW
    nchunk = e_per_w // _CHUNK
    src4 = edge_index[0].reshape(NW, nchunk // 2, 2, _CHUNK)
    dst3 = edge_index[1].reshape(NW, nchunk, _CHUNK)
    z_feat = jnp.zeros((n, d), jnp.float32)
    z_deg = jnp.zeros((n, 16), jnp.float32)

    agg1, deg = _sc_agg(x, src4, dst3, z_feat, z_deg, with_deg=True)
    h = _tc_dense(x, agg1, deg, W1_self, W1_neigh, b1, relu=True)
    (agg2,) = _sc_agg(h, src4, dst3, z_feat, z_deg, with_deg=False)
    out = _tc_dense(h, agg2, deg, W2_self, W2_neigh, b2, relu=False)
    return out


# speedup vs baseline: 12.8431x; 1.0254x over previous
import functools

import jax
import jax.numpy as jnp
from jax import lax
from jax.experimental import pallas as pl
from jax.experimental.pallas import tpu as pltpu
from jax.experimental.pallas import tpu_sc as plsc

NC = 2
NS = 16
NW = NC * NS

_CHUNK = 125


def _sc_agg(table, src4, dst4, z_feat, z_deg, *, with_deg):
    n, d = table.shape
    ngroup = src4.shape[1]
    rows_a = ((n + NS - 1) // NS + 7) // 8 * 8
    rows_last = n - rows_a * (NS - 1)
    mesh = plsc.VectorSubcoreMesh(
        core_axis_name="c", subcore_axis_name="s", num_cores=NC,
        num_subcores=NS)

    out_type = [jax.ShapeDtypeStruct((NC, n, d), jnp.float32)]
    scratch = [
        pltpu.VMEM((2, 2, _CHUNK), jnp.int32),
        pltpu.VMEM((2, 2, _CHUNK), jnp.int32),
        pltpu.VMEM((2, _CHUNK, d), jnp.float32),
        [pltpu.SemaphoreType.DMA] * 2,
        [pltpu.SemaphoreType.DMA] * 2,
        [pltpu.SemaphoreType.DMA] * 2,
        [pltpu.SemaphoreType.DMA] * 2,
        pltpu.VMEM_SHARED((n, d), jnp.float32),
    ]
    if with_deg:
        out_type.append(jax.ShapeDtypeStruct((NC, n, 16), jnp.float32))
        scratch += [
            pltpu.VMEM((_CHUNK, 16), jnp.float32),
            pltpu.VMEM_SHARED((n, 16), jnp.float32),
        ]

    @functools.partial(pl.kernel, out_type=out_type, mesh=mesh,
                       scratch_types=scratch,
                       compiler_params=pltpu.CompilerParams(
                           use_tc_tiling_on_sc=False))
    def k(table_hbm, src_hbm, dst_hbm, zf_hbm, zd_hbm, *refs):
        if with_deg:
            (agg_out, deg_out, sidx, didx, rows_v, gsem, ssem, isem, dsem,
             agg_sh, ones_v, deg_sh) = refs
        else:
            (agg_out, sidx, didx, rows_v, gsem, ssem, isem, dsem,
             agg_sh) = refs
        cid = lax.axis_index("c")
        sid = lax.axis_index("s")
        wid = cid * NS + sid

        pltpu.sync_copy(dst_hbm.at[wid, 0], didx.at[0])
        pltpu.async_copy(dst_hbm.at[wid, 1], didx.at[1], dsem[1])
        pltpu.sync_copy(src_hbm.at[wid, 0], sidx.at[0])
        pltpu.async_copy(src_hbm.at[wid, 1], sidx.at[1], isem[1])
        row0 = sid * rows_a

        def zero_slice(nrows):
            sl = pl.ds(row0, nrows)
            pltpu.sync_copy(zf_hbm.at[sl], agg_sh.at[sl])
            if with_deg:
                pltpu.sync_copy(zd_hbm.at[sl], deg_sh.at[sl])

        pl.when(sid < NS - 1)(lambda: zero_slice(rows_a))
        pl.when(sid == NS - 1)(lambda: zero_slice(rows_last))
        if with_deg:

            @pl.loop(0, _CHUNK)
            def _(r):
                ones_v[r, :] = jnp.ones((16,), jnp.float32)

        for b in range(2):
            pltpu.async_copy(table_hbm.at[sidx.at[0, b]], rows_v.at[b],
                             gsem[b])

        plsc.subcore_barrier()

        def step(g, G, H, b):
            pltpu.make_async_copy(table_hbm.at[sidx.at[0, b]],
                                  rows_v.at[b], gsem[b]).wait()
            if b == 0:
                @pl.when(g < ngroup - 1)
                def _():
                    pltpu.make_async_copy(src_hbm.at[wid, 0], sidx.at[H],
                                          isem[H]).wait()
                    pltpu.make_async_copy(dst_hbm.at[wid, 0], didx.at[H],
                                          dsem[H]).wait()
            pltpu.async_copy(rows_v.at[b], agg_sh.at[didx.at[G, b]],
                             ssem[b], add=True)
            if with_deg:
                pltpu.sync_copy(ones_v, deg_sh.at[didx.at[G, b]], add=True)
            if b == 1:
                @pl.when(g < ngroup - 2)
                def _():
                    pltpu.async_copy(src_hbm.at[wid, g + 2], sidx.at[G],
                                     isem[G])

            def drain_scatter():
                pltpu.make_async_copy(rows_v.at[b],
                                      agg_sh.at[didx.at[G, b]],
                                      ssem[b]).wait()

            @pl.when(g < ngroup - 1)
            def _():
                drain_scatter()
                pltpu.async_copy(table_hbm.at[sidx.at[H, b]],
                                 rows_v.at[b], gsem[b])
                if b == 1:
                    @pl.when(g < ngroup - 2)
                    def _():
                        pltpu.async_copy(dst_hbm.at[wid, g + 2],
                                         didx.at[G], dsem[G])

            pl.when(g == ngroup - 1)(drain_scatter)

        @pl.loop(0, ngroup // 2)
        def _(t):
            for b in range(2):
                step(2 * t, 0, 1, b)
            for b in range(2):
                step(2 * t + 1, 1, 0, b)

        plsc.subcore_barrier()

        def copy_out(nrows):
            sl = pl.ds(row0, nrows)
            pltpu.sync_copy(agg_sh.at[sl], agg_out.at[cid, sl])
            if with_deg:
                pltpu.sync_copy(deg_sh.at[sl], deg_out.at[cid, sl])

        pl.when(sid < NS - 1)(lambda: copy_out(rows_a))
        pl.when(sid == NS - 1)(lambda: copy_out(rows_last))

    return k(table, src4, dst4, z_feat, z_deg)


def _tc_dense(x, agg, deg, w_self, w_neigh, b, *, relu):
    n, d = x.shape
    r = 1000

    def body(x_ref, agg_ref, deg_ref, ws_ref, wn_ref, b_ref, o_ref):
        degs = deg_ref[0] + deg_ref[1]
        inv = 1.0 / jnp.maximum(degs[:, :1], 1.0)
        mean = (agg_ref[0] + agg_ref[1]) * inv
        out = (jnp.dot(x_ref[...], ws_ref[...],
                       preferred_element_type=jnp.float32)
               + jnp.dot(mean, wn_ref[...],
                         preferred_element_type=jnp.float32)
               + b_ref[...])
        o_ref[...] = jnp.maximum(out, 0.0) if relu else out

    return pl.pallas_call(
        body,
        grid=(n // r,),
        in_specs=[
            pl.BlockSpec((r, d), lambda i: (i, 0)),
            pl.BlockSpec((NC, r, d), lambda i: (0, i, 0)),
            pl.BlockSpec((NC, r, 16), lambda i: (0, i, 0)),
            pl.BlockSpec((d, d), lambda i: (0, 0)),
            pl.BlockSpec((d, d), lambda i: (0, 0)),
            pl.BlockSpec((1, d), lambda i: (0, 0)),
        ],
        out_specs=pl.BlockSpec((r, d), lambda i: (i, 0)),
        out_shape=jax.ShapeDtypeStruct((n, d), jnp.float32),
    )(x, agg, deg, w_self, w_neigh, b.reshape(1, d))


def kernel(x, edge_index, W1_self, W1_neigh, b1, W2_self, W2_neigh, b2):
    n, d = x.shape
    e = edge_index.shape[1]
    e_per_w = e // NW
    nchunk = e_per_w // _CHUNK
    src4 = edge_index[0].reshape(NW, nchunk // 2, 2, _CHUNK)
    dst4 = edge_index[1].reshape(NW, nchunk // 2, 2, _CHUNK)
    z_feat = jnp.zeros((n, d), jnp.float32)
    z_deg = jnp.zeros((n, 16), jnp.float32)

    agg1, deg = _sc_agg(x, src4, dst4, z_feat, z_deg, with_deg=True)
    h = _tc_dense(x, agg1, deg, W1_self, W1_neigh, b1, relu=True)
    (agg2,) = _sc_agg(h, src4, dst4, z_feat, z_deg, with_deg=False)
    out = _tc_dense(h, agg2, deg, W2_self, W2_neigh, b2, relu=False)
    return out


# speedup vs baseline: 12.9544x; 1.0087x over previous
import functools

import jax
import jax.numpy as jnp
from jax import lax
from jax.experimental import pallas as pl
from jax.experimental.pallas import tpu as pltpu
from jax.experimental.pallas import tpu_sc as plsc

NC = 2
NS = 16
NW = NC * NS

_CHUNK = 125


def _sc_agg(table, src4, dst4, z_feat, z_deg, *, with_deg):
    n, d = table.shape
    ngroup = src4.shape[1]
    rows_a = ((n + NS - 1) // NS + 7) // 8 * 8
    rows_last = n - rows_a * (NS - 1)
    mesh = plsc.VectorSubcoreMesh(
        core_axis_name="c", subcore_axis_name="s", num_cores=NC,
        num_subcores=NS)

    out_type = [jax.ShapeDtypeStruct((NC, n, d), jnp.float32)]
    scratch = [
        pltpu.VMEM((2, 2, _CHUNK), jnp.int32),
        pltpu.VMEM((2, 2, _CHUNK), jnp.int32),
        pltpu.VMEM((2, _CHUNK, d), jnp.float32),
        [pltpu.SemaphoreType.DMA] * 2,
        [pltpu.SemaphoreType.DMA] * 2,
        [pltpu.SemaphoreType.DMA] * 2,
        [pltpu.SemaphoreType.DMA] * 2,
        pltpu.VMEM_SHARED((n, d), jnp.float32),
    ]
    if with_deg:
        out_type.append(jax.ShapeDtypeStruct((NC, n, 16), jnp.float32))
        scratch += [
            pltpu.VMEM((_CHUNK, 16), jnp.float32),
            pltpu.VMEM_SHARED((n, 16), jnp.float32),
        ]

    @functools.partial(pl.kernel, out_type=out_type, mesh=mesh,
                       scratch_types=scratch,
                       compiler_params=pltpu.CompilerParams(
                           use_tc_tiling_on_sc=False))
    def k(table_hbm, src_hbm, dst_hbm, zf_hbm, zd_hbm, *refs):
        if with_deg:
            (agg_out, deg_out, sidx, didx, rows_v, gsem, ssem, isem, dsem,
             agg_sh, ones_v, deg_sh) = refs
        else:
            (agg_out, sidx, didx, rows_v, gsem, ssem, isem, dsem,
             agg_sh) = refs
        cid = lax.axis_index("c")
        sid = lax.axis_index("s")
        wid = cid * NS + sid

        pltpu.sync_copy(dst_hbm.at[wid, 0], didx.at[0])
        pltpu.async_copy(dst_hbm.at[wid, 1], didx.at[1], dsem[1])
        pltpu.sync_copy(src_hbm.at[wid, 0], sidx.at[0])
        pltpu.async_copy(src_hbm.at[wid, 1], sidx.at[1], isem[1])
        row0 = sid * rows_a

        def zero_slice(nrows):
            sl = pl.ds(row0, nrows)
            pltpu.sync_copy(zf_hbm.at[sl], agg_sh.at[sl])
            if with_deg:
                pltpu.sync_copy(zd_hbm.at[sl], deg_sh.at[sl])

        pl.when(sid < NS - 1)(lambda: zero_slice(rows_a))
        pl.when(sid == NS - 1)(lambda: zero_slice(rows_last))
        if with_deg:

            @pl.loop(0, _CHUNK)
            def _(r):
                ones_v[r, :] = jnp.ones((16,), jnp.float32)

        for b in range(2):
            pltpu.async_copy(table_hbm.at[sidx.at[0, b]], rows_v.at[b],
                             gsem[b])

        plsc.subcore_barrier()

        def step(g, G, H, b):
            pltpu.make_async_copy(table_hbm.at[sidx.at[0, b]],
                                  rows_v.at[b], gsem[b]).wait()
            if b == 0:
                @pl.when(g < ngroup - 1)
                def _():
                    pltpu.make_async_copy(src_hbm.at[wid, 0], sidx.at[H],
                                          isem[H]).wait()
                    pltpu.make_async_copy(dst_hbm.at[wid, 0], didx.at[H],
                                          dsem[H]).wait()
            pltpu.async_copy(rows_v.at[b], agg_sh.at[didx.at[G, b]],
                             ssem[b], add=True)
            if with_deg:
                pltpu.sync_copy(ones_v, deg_sh.at[didx.at[G, b]], add=True)
            if b == 1:
                @pl.when(g < ngroup - 2)
                def _():
                    pltpu.async_copy(src_hbm.at[wid, g + 2], sidx.at[G],
                                     isem[G])

            def drain_scatter():
                pltpu.make_async_copy(rows_v.at[b],
                                      agg_sh.at[didx.at[G, b]],
                                      ssem[b]).wait()

            @pl.when(g < ngroup - 1)
            def _():
                drain_scatter()
                pltpu.async_copy(table_hbm.at[sidx.at[H, b]],
                                 rows_v.at[b], gsem[b])
                if b == 1:
                    @pl.when(g < ngroup - 2)
                    def _():
                        pltpu.async_copy(dst_hbm.at[wid, g + 2],
                                         didx.at[G], dsem[G])

            pl.when(g == ngroup - 1)(drain_scatter)

        @pl.loop(0, ngroup // 2)
        def _(t):
            for b in range(2):
                step(2 * t, 0, 1, b)
            for b in range(2):
                step(2 * t + 1, 1, 0, b)

        plsc.subcore_barrier()

        def copy_out(nrows):
            sl = pl.ds(row0, nrows)
            pltpu.sync_copy(agg_sh.at[sl], agg_out.at[cid, sl])
            if with_deg:
                pltpu.sync_copy(deg_sh.at[sl], deg_out.at[cid, sl])

        pl.when(sid < NS - 1)(lambda: copy_out(rows_a))
        pl.when(sid == NS - 1)(lambda: copy_out(rows_last))

    return k(table, src4, dst4, z_feat, z_deg)


def _tc_self(x, w_self, b):
    n, d = x.shape
    r = 1000

    def body(x_ref, ws_ref, b_ref, o_ref):
        o_ref[...] = jnp.dot(x_ref[...], ws_ref[...],
                             preferred_element_type=jnp.float32) + b_ref[...]

    return pl.pallas_call(
        body,
        grid=(n // r,),
        in_specs=[
            pl.BlockSpec((r, d), lambda i: (i, 0)),
            pl.BlockSpec((d, d), lambda i: (0, 0)),
            pl.BlockSpec((1, d), lambda i: (0, 0)),
        ],
        out_specs=pl.BlockSpec((r, d), lambda i: (i, 0)),
        out_shape=jax.ShapeDtypeStruct((n, d), jnp.float32),
    )(x, w_self, b.reshape(1, d))


def _tc_combine(s, agg, deg, w_neigh, *, relu):
    n, d = s.shape
    r = 1000

    def body(s_ref, agg_ref, deg_ref, wn_ref, o_ref):
        degs = deg_ref[0] + deg_ref[1]
        inv = 1.0 / jnp.maximum(degs[:, :1], 1.0)
        mean = (agg_ref[0] + agg_ref[1]) * inv
        out = s_ref[...] + jnp.dot(mean, wn_ref[...],
                                   preferred_element_type=jnp.float32)
        o_ref[...] = jnp.maximum(out, 0.0) if relu else out

    return pl.pallas_call(
        body,
        grid=(n // r,),
        in_specs=[
            pl.BlockSpec((r, d), lambda i: (i, 0)),
            pl.BlockSpec((NC, r, d), lambda i: (0, i, 0)),
            pl.BlockSpec((NC, r, 16), lambda i: (0, i, 0)),
            pl.BlockSpec((d, d), lambda i: (0, 0)),
        ],
        out_specs=pl.BlockSpec((r, d), lambda i: (i, 0)),
        out_shape=jax.ShapeDtypeStruct((n, d), jnp.float32),
    )(s, agg, deg, w_neigh)


def kernel(x, edge_index, W1_self, W1_neigh, b1, W2_self, W2_neigh, b2):
    n, d = x.shape
    e = edge_index.shape[1]
    e_per_w = e // NW
    nchunk = e_per_w // _CHUNK
    src4 = edge_index[0].reshape(NW, nchunk // 2, 2, _CHUNK)
    dst4 = edge_index[1].reshape(NW, nchunk // 2, 2, _CHUNK)
    z_feat = jnp.zeros((n, d), jnp.float32)
    z_deg = jnp.zeros((n, 16), jnp.float32)

    agg1, deg = _sc_agg(x, src4, dst4, z_feat, z_deg, with_deg=True)
    s1 = _tc_self(x, W1_self, b1)
    h = _tc_combine(s1, agg1, deg, W1_neigh, relu=True)
    (agg2,) = _sc_agg(h, src4, dst4, z_feat, z_deg, with_deg=False)
    s2 = _tc_self(h, W2_self, b2)
    out = _tc_combine(s2, agg2, deg, W2_neigh, relu=False)
    return out
